# Initial kernel scaffold; baseline (speedup 1.0000x reference)
import jax
import jax.numpy as jnp
from jax.experimental import pallas as pl


def kernel(x, edge_index, edge_weight, W1, b1, a1, W2, b2, a2):
    raise NotImplementedError("write your pallas kernel here")


# speedup vs baseline: 15.6367x; 15.6367x over previous
import functools

import jax
import jax.numpy as jnp
from jax import lax
from jax.experimental import pallas as pl
from jax.experimental.pallas import tpu as pltpu
from jax.experimental.pallas import tpu_sc as plsc

N_NODES = 10000
N_PAD = 10240
D = 128

NC = 2
NS = 16
NW = NC * NS
E = 320000
EPW = E // NW
CH = 80
NCHUNK = EPW // CH
NG = 5
GC = 25

RPS = N_PAD // NS

_MESH = plsc.VectorSubcoreMesh(core_axis_name="c", subcore_axis_name="s")
_SC_PARAMS = pltpu.CompilerParams(needs_layout_passes=False)


def _wid():
    return lax.axis_index("s") * NC + lax.axis_index("c")


@functools.partial(
    pl.kernel,
    out_type=jax.ShapeDtypeStruct((NC, NS, RPS), jnp.float32),
    mesh=_MESH,
    compiler_params=_SC_PARAMS,
    scratch_types=[
        pltpu.VMEM((NCHUNK, CH), jnp.int32),
        pltpu.VMEM((NCHUNK, CH), jnp.float32),
        pltpu.VMEM((RPS,), jnp.float32),
        pltpu.VMEM_SHARED((N_PAD,), jnp.float32),
    ],
)
def _deg_kernel(col_hbm, ew_hbm, out_hbm, col_v, ew_v, buf_v, deg_sh):
    cid = lax.axis_index("c")
    sid = lax.axis_index("s")
    wid = _wid()

    zero16 = jnp.zeros((16,), jnp.float32)
    for i in range(RPS // 16):
        buf_v[pl.ds(i * 16, 16)] = zero16
    pltpu.sync_copy(buf_v, deg_sh.at[pl.ds(sid * RPS, RPS)])
    plsc.subcore_barrier()

    pltpu.sync_copy(col_hbm.at[wid], col_v)
    pltpu.sync_copy(ew_hbm.at[wid], ew_v)

    def chunk(t, carry):
        pltpu.sync_copy(ew_v.at[t], deg_sh.at[col_v.at[t]], add=True)
        return carry

    lax.fori_loop(0, NCHUNK, chunk, 0)
    plsc.subcore_barrier()

    pltpu.sync_copy(deg_sh.at[pl.ds(sid * RPS, RPS)], buf_v)
    pltpu.sync_copy(buf_v, out_hbm.at[cid, sid])


@functools.partial(
    pl.kernel,
    out_type=jax.ShapeDtypeStruct((NW, NCHUNK, CH), jnp.float32),
    mesh=_MESH,
    compiler_params=_SC_PARAMS,
    scratch_types=[
        pltpu.VMEM((NCHUNK, CH), jnp.int32),
        pltpu.VMEM((NCHUNK, CH), jnp.int32),
        pltpu.VMEM((NCHUNK, CH), jnp.float32),
        pltpu.VMEM((NCHUNK, CH), jnp.float32),
        pltpu.VMEM((N_PAD,), jnp.float32),
    ],
)
def _norm_kernel(row_hbm, col_hbm, ew_hbm, dis_hbm, out_hbm,
                 row_v, col_v, ew_v, nrm_v, dis_v):
    wid = _wid()
    pltpu.sync_copy(dis_hbm, dis_v)
    pltpu.sync_copy(row_hbm.at[wid], row_v)
    pltpu.sync_copy(col_hbm.at[wid], col_v)
    pltpu.sync_copy(ew_hbm.at[wid], ew_v)

    def chunk(t, carry):
        for k in range(CH // 16):
            sl = pl.ds(k * 16, 16)
            rv = row_v[t, sl]
            cv = col_v[t, sl]
            dr = plsc.load_gather(dis_v, [rv])
            dc = plsc.load_gather(dis_v, [cv])
            nrm_v[t, sl] = dr * ew_v[t, sl] * dc
        return carry

    lax.fori_loop(0, NCHUNK, chunk, 0)
    pltpu.sync_copy(nrm_v, out_hbm.at[wid])


@functools.partial(
    pl.kernel,
    out_type=jax.ShapeDtypeStruct((NC, NS, RPS, D), jnp.float32),
    mesh=_MESH,
    compiler_params=_SC_PARAMS,
    scratch_types=[
        pltpu.VMEM((GC, CH), jnp.int32),
        pltpu.VMEM((GC, CH), jnp.int32),
        pltpu.VMEM((GC, CH), jnp.float32),
        pltpu.VMEM((CH, D), jnp.float32),
        pltpu.VMEM_SHARED((N_PAD, D), jnp.float32),
        pltpu.SemaphoreType.DMA,
    ],
)
def _msg_kernel(h_hbm, row_hbm, col_hbm, nrm_hbm, out_hbm,
                row_v, col_v, nrm_v, rows_v, acc_sh, sem):
    cid = lax.axis_index("c")
    sid = lax.axis_index("s")
    wid = _wid()

    zero16 = jnp.zeros((16,), jnp.float32)

    def zrow(i, carry):
        for j in range(D // 16):
            rows_v[i, pl.ds(j * 16, 16)] = zero16
        return carry

    lax.fori_loop(0, CH, zrow, 0)
    for q in range(RPS // CH):
        pltpu.sync_copy(rows_v, acc_sh.at[pl.ds(sid * RPS + q * CH, CH)])
    plsc.subcore_barrier()

    def group(g, carry):
        pltpu.sync_copy(row_hbm.at[wid, g], row_v)
        pltpu.sync_copy(col_hbm.at[wid, g], col_v)
        pltpu.sync_copy(nrm_hbm.at[wid, g], nrm_v)

        def chunk(t, c2):
            pltpu.async_copy(h_hbm.at[row_v.at[t]], rows_v, sem).wait()
            for gg in range(CH // 16):
                nv = nrm_v[t, pl.ds(gg * 16, 16)]
                for k in range(16):
                    s = nv[k]
                    r = gg * 16 + k
                    for j in range(D // 16):
                        sl = pl.ds(j * 16, 16)
                        rows_v[r, sl] = rows_v[r, sl] * s
            pltpu.sync_copy(rows_v, acc_sh.at[col_v.at[t]], add=True)
            return c2

        lax.fori_loop(0, GC, chunk, 0)
        return carry

    lax.fori_loop(0, NG, group, 0)
    plsc.subcore_barrier()

    for q in range(RPS // CH):
        pltpu.sync_copy(acc_sh.at[pl.ds(sid * RPS + q * CH, CH)], rows_v)
        pltpu.sync_copy(rows_v, out_hbm.at[cid, sid, pl.ds(q * CH, CH)])


def _dis_body(degp_ref, dis_ref, inv_ref):
    deg = degp_ref[0] + degp_ref[1] + 1.0
    inv = 1.0 / deg
    inv_ref[...] = inv
    dis_ref[...] = jnp.sqrt(inv)


def _dis_call(degp):
    degp2 = degp.reshape(NC, N_PAD)
    return pl.pallas_call(
        _dis_body,
        out_shape=(
            jax.ShapeDtypeStruct((N_PAD,), jnp.float32),
            jax.ShapeDtypeStruct((N_PAD,), jnp.float32),
        ),
    )(degp2)


def _matmul_body(x_ref, w_ref, o_ref):
    o_ref[...] = jnp.dot(x_ref[...], w_ref[...],
                         preferred_element_type=jnp.float32)


def _matmul_call(x, w):
    bm = 1000
    grid = N_NODES // bm
    return pl.pallas_call(
        _matmul_body,
        grid=(grid,),
        in_specs=[
            pl.BlockSpec((bm, D), lambda i: (i, 0)),
            pl.BlockSpec((D, D), lambda i: (0, 0)),
        ],
        out_specs=pl.BlockSpec((bm, D), lambda i: (i, 0)),
        out_shape=jax.ShapeDtypeStruct((N_NODES, D), jnp.float32),
    )(x, w)


def _combine_mm_body(m0_ref, m1_ref, h_ref, inv_ref, b_ref, a_ref, w_ref,
                     o_ref):
    z = m0_ref[...] + m1_ref[...] + h_ref[...] * inv_ref[...] + b_ref[...]
    z = jnp.where(z > 0, z, a_ref[...] * z)
    o_ref[...] = jnp.dot(z, w_ref[...], preferred_element_type=jnp.float32)


def _combine_body(m0_ref, m1_ref, h_ref, inv_ref, b_ref, a_ref, o_ref):
    z = m0_ref[...] + m1_ref[...] + h_ref[...] * inv_ref[...] + b_ref[...]
    o_ref[...] = jnp.where(z > 0, z, a_ref[...] * z)


def _combine_call(m0, m1, h, inv_col, b, a, w=None):
    bm = 1000
    grid = N_NODES // bm
    node_spec = pl.BlockSpec((bm, D), lambda i: (i, 0))
    vec_spec = pl.BlockSpec((1, D), lambda i: (0, 0))
    in_specs = [node_spec, node_spec, node_spec,
                pl.BlockSpec((bm, 1), lambda i: (i, 0)),
                vec_spec, vec_spec]
    args = [m0, m1, h, inv_col, b.reshape(1, D), a.reshape(1, D)]
    if w is not None:
        in_specs.append(pl.BlockSpec((D, D), lambda i: (0, 0)))
        args.append(w)
        body = _combine_mm_body
    else:
        body = _combine_body
    return pl.pallas_call(
        body,
        grid=(grid,),
        in_specs=in_specs,
        out_specs=node_spec,
        out_shape=jax.ShapeDtypeStruct((N_NODES, D), jnp.float32),
    )(*args)


def kernel(x, edge_index, edge_weight, W1, b1, a1, W2, b2, a2):
    ei = edge_index.astype(jnp.int32)
    row3 = ei[0].reshape(NW, NCHUNK, CH)
    col3 = ei[1].reshape(NW, NCHUNK, CH)
    ew3 = edge_weight.reshape(NW, NCHUNK, CH)
    row4 = row3.reshape(NW, NG, GC, CH)
    col4 = col3.reshape(NW, NG, GC, CH)

    degp = _deg_kernel(col3, ew3)
    dis_flat, inv_flat = _dis_call(degp)
    inv_col = inv_flat[:N_NODES].reshape(N_NODES, 1)

    norm4 = _norm_kernel(row3, col3, ew3, dis_flat).reshape(NW, NG, GC, CH)
    h1 = _matmul_call(x, W1)

    m1 = _msg_kernel(h1, row4, col4, norm4).reshape(NC, N_PAD, D)[:, :N_NODES]
    h2 = _combine_call(m1[0], m1[1], h1, inv_col, b1, a1, w=W2)

    m2 = _msg_kernel(h2, row4, col4, norm4).reshape(NC, N_PAD, D)[:, :N_NODES]
    out = _combine_call(m2[0], m2[1], h2, inv_col, b2, a2)
    return out


# speedup vs baseline: 22.2469x; 1.4227x over previous
import functools

import jax
import jax.numpy as jnp
from jax import lax
from jax.experimental import pallas as pl
from jax.experimental.pallas import tpu as pltpu
from jax.experimental.pallas import tpu_sc as plsc

N_NODES = 10000
N_PAD = 10240
D = 128

NC = 2
NS = 16
NW = NC * NS
E = 320000
EPW = E // NW
CH = 80
NCHUNK = EPW // CH
NG = 5
GC = 25

RPS = N_PAD // NS

_MESH = plsc.VectorSubcoreMesh(core_axis_name="c", subcore_axis_name="s")
_SC_PARAMS = pltpu.CompilerParams(needs_layout_passes=False)


def _wid():
    return lax.axis_index("s") * NC + lax.axis_index("c")


@functools.partial(
    pl.kernel,
    out_type=jax.ShapeDtypeStruct((NC, NS, RPS), jnp.float32),
    mesh=_MESH,
    compiler_params=_SC_PARAMS,
    scratch_types=[
        pltpu.VMEM((NCHUNK, CH), jnp.int32),
        pltpu.VMEM((NCHUNK, CH), jnp.float32),
        pltpu.VMEM((RPS,), jnp.float32),
        pltpu.VMEM_SHARED((N_PAD,), jnp.float32),
    ],
)
def _deg_kernel(col_hbm, ew_hbm, out_hbm, col_v, ew_v, buf_v, deg_sh):
    cid = lax.axis_index("c")
    sid = lax.axis_index("s")
    wid = _wid()

    zero16 = jnp.zeros((16,), jnp.float32)
    for i in range(RPS // 16):
        buf_v[pl.ds(i * 16, 16)] = zero16
    pltpu.sync_copy(buf_v, deg_sh.at[pl.ds(sid * RPS, RPS)])
    plsc.subcore_barrier()

    pltpu.sync_copy(col_hbm.at[wid], col_v)
    pltpu.sync_copy(ew_hbm.at[wid], ew_v)

    def chunk(t, carry):
        pltpu.sync_copy(ew_v.at[t], deg_sh.at[col_v.at[t]], add=True)
        return carry

    lax.fori_loop(0, NCHUNK, chunk, 0)
    plsc.subcore_barrier()

    pltpu.sync_copy(deg_sh.at[pl.ds(sid * RPS, RPS)], buf_v)
    pltpu.sync_copy(buf_v, out_hbm.at[cid, sid])


@functools.partial(
    pl.kernel,
    out_type=jax.ShapeDtypeStruct((NW, NCHUNK, CH), jnp.float32),
    mesh=_MESH,
    compiler_params=_SC_PARAMS,
    scratch_types=[
        pltpu.VMEM((NCHUNK, CH), jnp.int32),
        pltpu.VMEM((NCHUNK, CH), jnp.int32),
        pltpu.VMEM((NCHUNK, CH), jnp.float32),
        pltpu.VMEM((NCHUNK, CH), jnp.float32),
        pltpu.VMEM((N_PAD,), jnp.float32),
    ],
)
def _norm_kernel(row_hbm, col_hbm, ew_hbm, dis_hbm, out_hbm,
                 row_v, col_v, ew_v, nrm_v, dis_v):
    wid = _wid()
    pltpu.sync_copy(dis_hbm, dis_v)
    pltpu.sync_copy(row_hbm.at[wid], row_v)
    pltpu.sync_copy(col_hbm.at[wid], col_v)
    pltpu.sync_copy(ew_hbm.at[wid], ew_v)

    def chunk(t, carry):
        for k in range(CH // 16):
            sl = pl.ds(k * 16, 16)
            rv = row_v[t, sl]
            cv = col_v[t, sl]
            dr = plsc.load_gather(dis_v, [rv])
            dc = plsc.load_gather(dis_v, [cv])
            nrm_v[t, sl] = dr * ew_v[t, sl] * dc
        return carry

    lax.fori_loop(0, NCHUNK, chunk, 0)
    pltpu.sync_copy(nrm_v, out_hbm.at[wid])


@functools.partial(
    pl.kernel,
    out_type=jax.ShapeDtypeStruct((NC, NS, RPS, D), jnp.float32),
    mesh=_MESH,
    compiler_params=_SC_PARAMS,
    scratch_types=[
        pltpu.VMEM((GC, CH), jnp.int32),
        pltpu.VMEM((GC, CH), jnp.int32),
        pltpu.VMEM((GC, CH), jnp.float32),
        pltpu.VMEM((CH, D), jnp.float32),
        pltpu.VMEM((CH, D), jnp.float32),
        pltpu.VMEM_SHARED((N_PAD, D), jnp.float32),
        pltpu.SemaphoreType.DMA,
        pltpu.SemaphoreType.DMA,
        pltpu.SemaphoreType.DMA,
        pltpu.SemaphoreType.DMA,
    ],
)
def _msg_kernel(h_hbm, row_hbm, col_hbm, nrm_hbm, out_hbm,
                row_v, col_v, nrm_v, buf_a, buf_b, acc_sh,
                gsem_a, gsem_b, ssem_a, ssem_b):
    cid = lax.axis_index("c")
    sid = lax.axis_index("s")
    wid = _wid()

    zero16 = jnp.zeros((16,), jnp.float32)

    def zrow(i, carry):
        for j in range(D // 16):
            buf_a[i, pl.ds(j * 16, 16)] = zero16
        return carry

    lax.fori_loop(0, CH, zrow, 0)
    for q in range(RPS // CH):
        pltpu.sync_copy(buf_a, acc_sh.at[pl.ds(sid * RPS + q * CH, CH)])
    plsc.subcore_barrier()

    def scale(buf, t):
        def gg_body(gg, carry):
            nv = nrm_v[t, pl.ds(gg * 16, 16)]
            for k in range(16):
                s = nv[k]
                r = gg * 16 + k
                for j in range(D // 16):
                    sl = pl.ds(j * 16, 16)
                    buf[r, sl] = buf[r, sl] * s
            return carry
        lax.fori_loop(0, CH // 16, gg_body, 0)

    def gather_start(t, buf, sem):
        pltpu.make_async_copy(h_hbm.at[row_v.at[t]], buf, sem).start()

    def gather_wait(t, buf, sem):
        pltpu.make_async_copy(h_hbm.at[row_v.at[t]], buf, sem).wait()

    def scatter_start(t, buf, sem):
        pltpu.make_async_copy(buf, acc_sh.at[col_v.at[t]], sem).start(add=True)

    def scatter_wait(t, buf, sem):
        pltpu.make_async_copy(buf, acc_sh.at[col_v.at[t]], sem).wait()

    def group(g, carry):
        pltpu.sync_copy(row_hbm.at[wid, g], row_v)
        pltpu.sync_copy(col_hbm.at[wid, g], col_v)
        pltpu.sync_copy(nrm_hbm.at[wid, g], nrm_v)

        gather_start(0, buf_a, gsem_a)

        def pair(i, c2):
            t0 = 2 * i
            t1 = t0 + 1
            gather_wait(t0, buf_a, gsem_a)

            @pl.when(i > 0)
            def _():
                scatter_wait(t0 - 1, buf_b, ssem_b)

            gather_start(t1, buf_b, gsem_b)
            scale(buf_a, t0)
            scatter_start(t0, buf_a, ssem_a)
            gather_wait(t1, buf_b, gsem_b)
            scatter_wait(t0, buf_a, ssem_a)
            gather_start(t1 + 1, buf_a, gsem_a)
            scale(buf_b, t1)
            scatter_start(t1, buf_b, ssem_b)
            return c2

        lax.fori_loop(0, GC // 2, pair, 0)
        tl = GC - 1
        gather_wait(tl, buf_a, gsem_a)
        scatter_wait(tl - 1, buf_b, ssem_b)
        scale(buf_a, tl)
        scatter_start(tl, buf_a, ssem_a)
        scatter_wait(tl, buf_a, ssem_a)
        return carry

    lax.fori_loop(0, NG, group, 0)
    plsc.subcore_barrier()

    for q in range(RPS // CH):
        pltpu.sync_copy(acc_sh.at[pl.ds(sid * RPS + q * CH, CH)], buf_a)
        pltpu.sync_copy(buf_a, out_hbm.at[cid, sid, pl.ds(q * CH, CH)])


def _dis_body(degp_ref, dis_ref, inv_ref):
    deg = degp_ref[0] + degp_ref[1] + 1.0
    inv = 1.0 / deg
    inv_ref[...] = inv
    dis_ref[...] = jnp.sqrt(inv)


def _dis_call(degp):
    degp2 = degp.reshape(NC, N_PAD)
    return pl.pallas_call(
        _dis_body,
        out_shape=(
            jax.ShapeDtypeStruct((N_PAD,), jnp.float32),
            jax.ShapeDtypeStruct((N_PAD,), jnp.float32),
        ),
    )(degp2)


def _matmul_body(x_ref, w_ref, o_ref):
    o_ref[...] = jnp.dot(x_ref[...], w_ref[...],
                         preferred_element_type=jnp.float32)


def _matmul_call(x, w):
    bm = 1000
    grid = N_NODES // bm
    return pl.pallas_call(
        _matmul_body,
        grid=(grid,),
        in_specs=[
            pl.BlockSpec((bm, D), lambda i: (i, 0)),
            pl.BlockSpec((D, D), lambda i: (0, 0)),
        ],
        out_specs=pl.BlockSpec((bm, D), lambda i: (i, 0)),
        out_shape=jax.ShapeDtypeStruct((N_NODES, D), jnp.float32),
    )(x, w)


def _combine_mm_body(m0_ref, m1_ref, h_ref, inv_ref, b_ref, a_ref, w_ref,
                     o_ref):
    z = m0_ref[...] + m1_ref[...] + h_ref[...] * inv_ref[...] + b_ref[...]
    z = jnp.where(z > 0, z, a_ref[...] * z)
    o_ref[...] = jnp.dot(z, w_ref[...], preferred_element_type=jnp.float32)


def _combine_body(m0_ref, m1_ref, h_ref, inv_ref, b_ref, a_ref, o_ref):
    z = m0_ref[...] + m1_ref[...] + h_ref[...] * inv_ref[...] + b_ref[...]
    o_ref[...] = jnp.where(z > 0, z, a_ref[...] * z)


def _combine_call(m0, m1, h, inv_col, b, a, w=None):
    bm = 1000
    grid = N_NODES // bm
    node_spec = pl.BlockSpec((bm, D), lambda i: (i, 0))
    vec_spec = pl.BlockSpec((1, D), lambda i: (0, 0))
    in_specs = [node_spec, node_spec, node_spec,
                pl.BlockSpec((bm, 1), lambda i: (i, 0)),
                vec_spec, vec_spec]
    args = [m0, m1, h, inv_col, b.reshape(1, D), a.reshape(1, D)]
    if w is not None:
        in_specs.append(pl.BlockSpec((D, D), lambda i: (0, 0)))
        args.append(w)
        body = _combine_mm_body
    else:
        body = _combine_body
    return pl.pallas_call(
        body,
        grid=(grid,),
        in_specs=in_specs,
        out_specs=node_spec,
        out_shape=jax.ShapeDtypeStruct((N_NODES, D), jnp.float32),
    )(*args)


def kernel(x, edge_index, edge_weight, W1, b1, a1, W2, b2, a2):
    ei = edge_index.astype(jnp.int32)
    row3 = ei[0].reshape(NW, NCHUNK, CH)
    col3 = ei[1].reshape(NW, NCHUNK, CH)
    ew3 = edge_weight.reshape(NW, NCHUNK, CH)
    row4 = row3.reshape(NW, NG, GC, CH)
    col4 = col3.reshape(NW, NG, GC, CH)

    degp = _deg_kernel(col3, ew3)
    dis_flat, inv_flat = _dis_call(degp)
    inv_col = inv_flat[:N_NODES].reshape(N_NODES, 1)

    norm4 = _norm_kernel(row3, col3, ew3, dis_flat).reshape(NW, NG, GC, CH)
    h1 = _matmul_call(x, W1)

    m1 = _msg_kernel(h1, row4, col4, norm4).reshape(NC, N_PAD, D)[:, :N_NODES]
    h2 = _combine_call(m1[0], m1[1], h1, inv_col, b1, a1, w=W2)

    m2 = _msg_kernel(h2, row4, col4, norm4).reshape(NC, N_PAD, D)[:, :N_NODES]
    out = _combine_call(m2[0], m2[1], h2, inv_col, b2, a2)
    return out
